# Initial kernel scaffold; baseline (speedup 1.0000x reference)
import jax
import jax.numpy as jnp
from jax.experimental import pallas as pl


def kernel(x, edge_index, W1, b1, W2, b2, W3, b3):
    raise NotImplementedError("write your pallas kernel here")


# speedup vs baseline: 31.5336x; 31.5336x over previous
import jax
import jax.numpy as jnp
from jax import lax
from jax.experimental import pallas as pl
from jax.experimental.pallas import tpu as pltpu
from jax.experimental.pallas import tpu_sc as plsc

_N = 10000
_E = 320000
_NC = 2
_NS = 16
_CH = 125
_R = _E // _CH
_TPW = _R // (_NC * _NS)
_NACC = 10240
_NPT = _NACC // _NS
_ZROWS = 128


def _sc_mesh():
    return plsc.VectorSubcoreMesh(core_axis_name="c", subcore_axis_name="s")


def _make_agg(F):
    def body(u_hbm, src_hbm, dst_hbm, out_hbm,
             src_v, dst_v, rows0, rows1, zbuf, acc, sem0, sem1):
        c = lax.axis_index("c")
        s = lax.axis_index("s")
        tile = c * _NS + s
        base = tile * _TPW
        pltpu.sync_copy(src_hbm.at[pl.ds(base, _TPW)], src_v)
        pltpu.sync_copy(dst_hbm.at[pl.ds(base, _TPW)], dst_v)

        zero = jnp.zeros((16,), jnp.float32)

        def zrow(i, carry):
            for kk in range(F // 16):
                zbuf[i, pl.ds(kk * 16, 16)] = zero
            return carry

        lax.fori_loop(0, _ZROWS, zrow, 0)
        for r in range(_NPT // _ZROWS):
            pltpu.sync_copy(zbuf, acc.at[pl.ds(s * _NPT + r * _ZROWS, _ZROWS)])
        plsc.subcore_barrier()

        pltpu.async_copy(u_hbm.at[src_v.at[0]], rows0, sem0)
        pltpu.async_copy(u_hbm.at[src_v.at[1]], rows1, sem1)

        def wait0():
            pltpu.make_async_copy(u_hbm.at[src_v.at[0]], rows0, sem0).wait()

        def wait1():
            pltpu.make_async_copy(u_hbm.at[src_v.at[1]], rows1, sem1).wait()

        def loop_body(i, carry):
            j0 = 2 * i
            wait0()
            pltpu.sync_copy(rows0, acc.at[dst_v.at[j0]], add=True)
            pltpu.async_copy(u_hbm.at[src_v.at[j0 + 2]], rows0, sem0)
            wait1()
            pltpu.sync_copy(rows1, acc.at[dst_v.at[j0 + 1]], add=True)
            pltpu.async_copy(u_hbm.at[src_v.at[j0 + 3]], rows1, sem1)
            return carry

        lax.fori_loop(0, _TPW // 2 - 1, loop_body, 0)
        wait0()
        pltpu.sync_copy(rows0, acc.at[dst_v.at[_TPW - 2]], add=True)
        wait1()
        pltpu.sync_copy(rows1, acc.at[dst_v.at[_TPW - 1]], add=True)
        plsc.subcore_barrier()
        for r in range(_NPT // _ZROWS):
            off = s * _NPT + r * _ZROWS
            pltpu.sync_copy(acc.at[pl.ds(off, _ZROWS)],
                            out_hbm.at[c].at[pl.ds(off, _ZROWS)])

    return pl.kernel(
        body,
        out_type=jax.ShapeDtypeStruct((_NC, _NACC, F), jnp.float32),
        mesh=_sc_mesh(),
        compiler_params=pltpu.CompilerParams(use_tc_tiling_on_sc=False),
        scratch_types=[
            pltpu.VMEM((_TPW, _CH), jnp.int32),
            pltpu.VMEM((_TPW, _CH), jnp.int32),
            pltpu.VMEM((_CH, F), jnp.float32),
            pltpu.VMEM((_CH, F), jnp.float32),
            pltpu.VMEM((_ZROWS, F), jnp.float32),
            pltpu.VMEM_SHARED((_NACC, F), jnp.float32),
            pltpu.SemaphoreType.DMA,
            pltpu.SemaphoreType.DMA,
        ],
    )


def _make_deg():
    F = 16

    def body(dst_hbm, out_hbm, dst_v, ones_v, zbuf, acc):
        c = lax.axis_index("c")
        s = lax.axis_index("s")
        tile = c * _NS + s
        base = tile * _TPW
        pltpu.sync_copy(dst_hbm.at[pl.ds(base, _TPW)], dst_v)

        zero = jnp.zeros((16,), jnp.float32)
        one = jnp.ones((16,), jnp.float32)

        def fill_z(i, carry):
            zbuf[i, :] = zero
            return carry

        lax.fori_loop(0, _ZROWS, fill_z, 0)

        def fill_o(i, carry):
            ones_v[i, :] = one
            return carry

        lax.fori_loop(0, _CH, fill_o, 0)
        for r in range(_NPT // _ZROWS):
            pltpu.sync_copy(zbuf, acc.at[pl.ds(s * _NPT + r * _ZROWS, _ZROWS)])
        plsc.subcore_barrier()

        def loop_body(j, carry):
            pltpu.sync_copy(ones_v, acc.at[dst_v.at[j]], add=True)
            return carry

        lax.fori_loop(0, _TPW, loop_body, 0)
        plsc.subcore_barrier()
        for r in range(_NPT // _ZROWS):
            off = s * _NPT + r * _ZROWS
            pltpu.sync_copy(acc.at[pl.ds(off, _ZROWS)],
                            out_hbm.at[c].at[pl.ds(off, _ZROWS)])

    return pl.kernel(
        body,
        out_type=jax.ShapeDtypeStruct((_NC, _NACC, F), jnp.float32),
        mesh=_sc_mesh(),
        compiler_params=pltpu.CompilerParams(use_tc_tiling_on_sc=False),
        scratch_types=[
            pltpu.VMEM((_TPW, _CH), jnp.int32),
            pltpu.VMEM((_CH, F), jnp.float32),
            pltpu.VMEM((_ZROWS, F), jnp.float32),
            pltpu.VMEM_SHARED((_NACC, F), jnp.float32),
        ],
    )


_BN = 1000


def _tc_first(degp, x, w1p):
    fin, fout = x.shape[1], w1p.shape[1]

    def body(degp_ref, x_ref, w_ref, u_ref, dinv_ref):
        deg = jnp.maximum(degp_ref[0] + degp_ref[1], 1.0)
        r = lax.rsqrt(deg)
        dinv = r * (1.5 - 0.5 * deg * r * r)
        dinv_ref[...] = dinv
        xw = jnp.dot(x_ref[...], w_ref[...],
                     preferred_element_type=jnp.float32)
        u_ref[...] = xw * dinv[:, 0:1]

    return pl.pallas_call(
        body,
        grid=(_N // _BN,),
        in_specs=[
            pl.BlockSpec((_NC, _BN, 16), lambda i: (0, i, 0)),
            pl.BlockSpec((_BN, fin), lambda i: (i, 0)),
            pl.BlockSpec((fin, fout), lambda i: (0, 0)),
        ],
        out_specs=[
            pl.BlockSpec((_BN, fout), lambda i: (i, 0)),
            pl.BlockSpec((_BN, 16), lambda i: (i, 0)),
        ],
        out_shape=[
            jax.ShapeDtypeStruct((_N, fout), jnp.float32),
            jax.ShapeDtypeStruct((_N, 16), jnp.float32),
        ],
    )(degp, x, w1p)


def _tc_mid(sp, dinv, wp, bp):
    fin, fout = wp.shape

    def body(sp_ref, dinv_ref, w_ref, b_ref, u_ref):
        dinv = dinv_ref[...]
        h = jnp.maximum(dinv[:, 0:1] * (sp_ref[0] + sp_ref[1]) + b_ref[...],
                        0.0)
        hw = jnp.dot(h, w_ref[...], preferred_element_type=jnp.float32)
        u_ref[...] = hw * dinv[:, 0:1]

    return pl.pallas_call(
        body,
        grid=(_N // _BN,),
        in_specs=[
            pl.BlockSpec((_NC, _BN, fin), lambda i: (0, i, 0)),
            pl.BlockSpec((_BN, 16), lambda i: (i, 0)),
            pl.BlockSpec((fin, fout), lambda i: (0, 0)),
            pl.BlockSpec((1, fin), lambda i: (0, 0)),
        ],
        out_specs=pl.BlockSpec((_BN, fout), lambda i: (i, 0)),
        out_shape=jax.ShapeDtypeStruct((_N, fout), jnp.float32),
    )(sp, dinv, wp, bp)


def _tc_last(sp, dinv, bp):
    fin = sp.shape[2]

    def body(sp_ref, dinv_ref, b_ref, o_ref):
        dinv = dinv_ref[...]
        o_ref[...] = dinv[:, 0:1] * (sp_ref[0] + sp_ref[1]) + b_ref[...]

    return pl.pallas_call(
        body,
        grid=(_N // _BN,),
        in_specs=[
            pl.BlockSpec((_NC, _BN, fin), lambda i: (0, i, 0)),
            pl.BlockSpec((_BN, 16), lambda i: (i, 0)),
            pl.BlockSpec((1, fin), lambda i: (0, 0)),
        ],
        out_specs=pl.BlockSpec((_BN, fin), lambda i: (i, 0)),
        out_shape=jax.ShapeDtypeStruct((_N, fin), jnp.float32),
    )(sp, dinv, bp)


_agg32 = _make_agg(32)
_agg48 = _make_agg(48)
_agg16 = _make_agg(16)
_deg = _make_deg()


def kernel(x, edge_index, W1, b1, W2, b2, W3, b3):
    src2d = edge_index[0].reshape(_R, _CH)
    dst2d = edge_index[1].reshape(_R, _CH)

    w1p = jnp.pad(W1, ((0, 0), (0, 32 - W1.shape[1])))
    w2p = jnp.pad(W2, ((0, 32 - W2.shape[0]), (0, 48 - W2.shape[1])))
    w3p = jnp.pad(W3, ((0, 48 - W3.shape[0]), (0, 16 - W3.shape[1])))
    b1p = jnp.pad(b1, (0, 32 - b1.shape[0])).reshape(1, 32)
    b2p = jnp.pad(b2, (0, 48 - b2.shape[0])).reshape(1, 48)
    b3p = jnp.pad(b3, (0, 16 - b3.shape[0])).reshape(1, 16)

    degp = _deg(dst2d)
    u1, dinv = _tc_first(degp, x, w1p)
    s1p = _agg32(u1, src2d, dst2d)
    u2 = _tc_mid(s1p, dinv, w2p, b1p)
    s2p = _agg48(u2, src2d, dst2d)
    u3 = _tc_mid(s2p, dinv, w3p, b2p)
    s3p = _agg16(u3, src2d, dst2d)
    out16 = _tc_last(s3p, dinv, b3p)
    return out16[:, :2]


# speedup vs baseline: 36.7017x; 1.1639x over previous
import jax
import jax.numpy as jnp
from jax import lax
from jax.experimental import pallas as pl
from jax.experimental.pallas import tpu as pltpu
from jax.experimental.pallas import tpu_sc as plsc

_N = 10000
_E = 320000
_NC = 2
_NS = 16
_CH = 125
_R = _E // _CH
_TPW = _R // (_NC * _NS)
_NACC = 10240
_NPT = _NACC // _NS
_ZROWS = 128


def _sc_mesh():
    return plsc.VectorSubcoreMesh(core_axis_name="c", subcore_axis_name="s")


def _make_agg(F):
    def body(u_hbm, src_hbm, dst_hbm, out_hbm,
             src_v, dst_v, rows0, rows1, rows2, rows3, zbuf, acc,
             sg0, sg1, sg2, sg3, ss0, ss1, ss2, ss3):
        c = lax.axis_index("c")
        s = lax.axis_index("s")
        tile = c * _NS + s
        base = tile * _TPW
        pltpu.sync_copy(src_hbm.at[pl.ds(base, _TPW)], src_v)
        pltpu.sync_copy(dst_hbm.at[pl.ds(base, _TPW)], dst_v)

        bufs = (rows0, rows1, rows2, rows3)
        sg = (sg0, sg1, sg2, sg3)
        ss = (ss0, ss1, ss2, ss3)

        zero = jnp.zeros((16,), jnp.float32)

        def zrow(i, carry):
            for kk in range(F // 16):
                zbuf[i, pl.ds(kk * 16, 16)] = zero
            return carry

        lax.fori_loop(0, _ZROWS, zrow, 0)
        for r in range(_NPT // _ZROWS):
            pltpu.sync_copy(zbuf, acc.at[pl.ds(s * _NPT + r * _ZROWS, _ZROWS)])
        plsc.subcore_barrier()

        def wait_g(b):
            pltpu.make_async_copy(u_hbm.at[src_v.at[0]], bufs[b], sg[b]).wait()

        def wait_s(b):
            pltpu.make_async_copy(bufs[b], acc.at[dst_v.at[0]], ss[b]).wait()

        for b in range(4):
            pltpu.async_copy(u_hbm.at[src_v.at[b]], bufs[b], sg[b])

        nout = _TPW // 4

        def loop_body(i, carry):
            j0 = 4 * i
            for b in range(4):
                wait_g(b)
                pltpu.async_copy(bufs[b], acc.at[dst_v.at[j0 + b]], ss[b],
                                 add=True)

            @pl.when(i < nout - 1)
            def _():
                for b in range(4):
                    wait_s(b)
                    pltpu.async_copy(u_hbm.at[src_v.at[j0 + 4 + b]],
                                     bufs[b], sg[b])

            return carry

        lax.fori_loop(0, nout, loop_body, 0)
        for b in range(4):
            wait_s(b)
        plsc.subcore_barrier()
        for r in range(_NPT // _ZROWS):
            off = s * _NPT + r * _ZROWS
            pltpu.sync_copy(acc.at[pl.ds(off, _ZROWS)],
                            out_hbm.at[c].at[pl.ds(off, _ZROWS)])

    return pl.kernel(
        body,
        out_type=jax.ShapeDtypeStruct((_NC, _NACC, F), jnp.float32),
        mesh=_sc_mesh(),
        compiler_params=pltpu.CompilerParams(use_tc_tiling_on_sc=False),
        scratch_types=[
            pltpu.VMEM((_TPW, _CH), jnp.int32),
            pltpu.VMEM((_TPW, _CH), jnp.int32),
            pltpu.VMEM((_CH, F), jnp.float32),
            pltpu.VMEM((_CH, F), jnp.float32),
            pltpu.VMEM((_CH, F), jnp.float32),
            pltpu.VMEM((_CH, F), jnp.float32),
            pltpu.VMEM((_ZROWS, F), jnp.float32),
            pltpu.VMEM_SHARED((_NACC, F), jnp.float32),
            pltpu.SemaphoreType.DMA,
            pltpu.SemaphoreType.DMA,
            pltpu.SemaphoreType.DMA,
            pltpu.SemaphoreType.DMA,
            pltpu.SemaphoreType.DMA,
            pltpu.SemaphoreType.DMA,
            pltpu.SemaphoreType.DMA,
            pltpu.SemaphoreType.DMA,
        ],
    )


def _make_deg():
    F = 16

    def body(dst_hbm, out_hbm, dst_v, ones_v, zbuf, acc, ss0, ss1, ss2, ss3):
        c = lax.axis_index("c")
        s = lax.axis_index("s")
        tile = c * _NS + s
        base = tile * _TPW
        pltpu.sync_copy(dst_hbm.at[pl.ds(base, _TPW)], dst_v)

        ss = (ss0, ss1, ss2, ss3)
        zero = jnp.zeros((16,), jnp.float32)
        one = jnp.ones((16,), jnp.float32)

        def fill_z(i, carry):
            zbuf[i, :] = zero
            return carry

        lax.fori_loop(0, _ZROWS, fill_z, 0)

        def fill_o(i, carry):
            ones_v[i, :] = one
            return carry

        lax.fori_loop(0, _CH, fill_o, 0)
        for r in range(_NPT // _ZROWS):
            pltpu.sync_copy(zbuf, acc.at[pl.ds(s * _NPT + r * _ZROWS, _ZROWS)])
        plsc.subcore_barrier()

        def wait_s(b):
            pltpu.make_async_copy(ones_v.at[pl.ds(0, _CH)],
                                  acc.at[dst_v.at[0]], ss[b]).wait()

        for b in range(4):
            pltpu.async_copy(ones_v.at[pl.ds(0, _CH)], acc.at[dst_v.at[b]],
                             ss[b], add=True)

        def loop_body(i, carry):
            j0 = 4 * (i + 1)
            for b in range(4):
                wait_s(b)
                pltpu.async_copy(ones_v.at[pl.ds(0, _CH)],
                                 acc.at[dst_v.at[j0 + b]], ss[b], add=True)
            return carry

        lax.fori_loop(0, _TPW // 4 - 1, loop_body, 0)
        for b in range(4):
            wait_s(b)
        plsc.subcore_barrier()
        for r in range(_NPT // _ZROWS):
            off = s * _NPT + r * _ZROWS
            pltpu.sync_copy(acc.at[pl.ds(off, _ZROWS)],
                            out_hbm.at[c].at[pl.ds(off, _ZROWS)])

    return pl.kernel(
        body,
        out_type=jax.ShapeDtypeStruct((_NC, _NACC, F), jnp.float32),
        mesh=_sc_mesh(),
        compiler_params=pltpu.CompilerParams(use_tc_tiling_on_sc=False),
        scratch_types=[
            pltpu.VMEM((_TPW, _CH), jnp.int32),
            pltpu.VMEM((_CH, F), jnp.float32),
            pltpu.VMEM((_ZROWS, F), jnp.float32),
            pltpu.VMEM_SHARED((_NACC, F), jnp.float32),
            pltpu.SemaphoreType.DMA,
            pltpu.SemaphoreType.DMA,
            pltpu.SemaphoreType.DMA,
            pltpu.SemaphoreType.DMA,
        ],
    )


_BN = 1000


def _tc_first(degp, x, w1p):
    fin, fout = x.shape[1], w1p.shape[1]

    def body(degp_ref, x_ref, w_ref, u_ref, dinv_ref):
        deg = jnp.maximum(degp_ref[0] + degp_ref[1], 1.0)
        r = lax.rsqrt(deg)
        dinv = r * (1.5 - 0.5 * deg * r * r)
        dinv_ref[...] = dinv
        xw = jnp.dot(x_ref[...], w_ref[...],
                     preferred_element_type=jnp.float32)
        u_ref[...] = xw * dinv[:, 0:1]

    return pl.pallas_call(
        body,
        grid=(_N // _BN,),
        in_specs=[
            pl.BlockSpec((_NC, _BN, 16), lambda i: (0, i, 0)),
            pl.BlockSpec((_BN, fin), lambda i: (i, 0)),
            pl.BlockSpec((fin, fout), lambda i: (0, 0)),
        ],
        out_specs=[
            pl.BlockSpec((_BN, fout), lambda i: (i, 0)),
            pl.BlockSpec((_BN, 16), lambda i: (i, 0)),
        ],
        out_shape=[
            jax.ShapeDtypeStruct((_N, fout), jnp.float32),
            jax.ShapeDtypeStruct((_N, 16), jnp.float32),
        ],
    )(degp, x, w1p)


def _tc_mid(sp, dinv, wp, bp):
    fin, fout = wp.shape

    def body(sp_ref, dinv_ref, w_ref, b_ref, u_ref):
        dinv = dinv_ref[...]
        h = jnp.maximum(dinv[:, 0:1] * (sp_ref[0] + sp_ref[1]) + b_ref[...],
                        0.0)
        hw = jnp.dot(h, w_ref[...], preferred_element_type=jnp.float32)
        u_ref[...] = hw * dinv[:, 0:1]

    return pl.pallas_call(
        body,
        grid=(_N // _BN,),
        in_specs=[
            pl.BlockSpec((_NC, _BN, fin), lambda i: (0, i, 0)),
            pl.BlockSpec((_BN, 16), lambda i: (i, 0)),
            pl.BlockSpec((fin, fout), lambda i: (0, 0)),
            pl.BlockSpec((1, fin), lambda i: (0, 0)),
        ],
        out_specs=pl.BlockSpec((_BN, fout), lambda i: (i, 0)),
        out_shape=jax.ShapeDtypeStruct((_N, fout), jnp.float32),
    )(sp, dinv, wp, bp)


def _tc_last(sp, dinv, bp):
    fin = sp.shape[2]

    def body(sp_ref, dinv_ref, b_ref, o_ref):
        dinv = dinv_ref[...]
        o_ref[...] = dinv[:, 0:1] * (sp_ref[0] + sp_ref[1]) + b_ref[...]

    return pl.pallas_call(
        body,
        grid=(_N // _BN,),
        in_specs=[
            pl.BlockSpec((_NC, _BN, fin), lambda i: (0, i, 0)),
            pl.BlockSpec((_BN, 16), lambda i: (i, 0)),
            pl.BlockSpec((1, fin), lambda i: (0, 0)),
        ],
        out_specs=pl.BlockSpec((_BN, fin), lambda i: (i, 0)),
        out_shape=jax.ShapeDtypeStruct((_N, fin), jnp.float32),
    )(sp, dinv, bp)


_agg32 = _make_agg(32)
_agg48 = _make_agg(48)
_agg16 = _make_agg(16)
_deg = _make_deg()


def kernel(x, edge_index, W1, b1, W2, b2, W3, b3):
    src2d = edge_index[0].reshape(_R, _CH)
    dst2d = edge_index[1].reshape(_R, _CH)

    w1p = jnp.pad(W1, ((0, 0), (0, 32 - W1.shape[1])))
    w2p = jnp.pad(W2, ((0, 32 - W2.shape[0]), (0, 48 - W2.shape[1])))
    w3p = jnp.pad(W3, ((0, 48 - W3.shape[0]), (0, 16 - W3.shape[1])))
    b1p = jnp.pad(b1, (0, 32 - b1.shape[0])).reshape(1, 32)
    b2p = jnp.pad(b2, (0, 48 - b2.shape[0])).reshape(1, 48)
    b3p = jnp.pad(b3, (0, 16 - b3.shape[0])).reshape(1, 16)

    degp = _deg(dst2d)
    u1, dinv = _tc_first(degp, x, w1p)
    s1p = _agg32(u1, src2d, dst2d)
    u2 = _tc_mid(s1p, dinv, w2p, b1p)
    s2p = _agg48(u2, src2d, dst2d)
    u3 = _tc_mid(s2p, dinv, w3p, b2p)
    s3p = _agg16(u3, src2d, dst2d)
    out16 = _tc_last(s3p, dinv, b3p)
    return out16[:, :2]
